# Initial kernel scaffold; baseline (speedup 1.0000x reference)
import functools

import jax
import jax.numpy as jnp
from jax import lax
from jax.experimental import pallas as pl
from jax.experimental.pallas import tpu as pltpu
from jax.experimental.pallas import tpu_sc as plsc

NC = 2
NS = 16
NW = NC * NS
LANES = 16
D = 128
CHUNK = 128


def _embed_body(idx_hbm, table_hbm, out_hbm, *, bag, chunks_per_tile, wid,
                idx_v, rows_v, out_v, sem):
    nrows = bag * CHUNK
    nbatch = nrows // CHUNK

    def chunk_body(c, _):
        bag0 = wid * (chunks_per_tile * CHUNK) + c * CHUNK
        irow0 = bag0 * bag // CHUNK
        pltpu.sync_copy(idx_hbm.at[pl.ds(irow0, nbatch)], idx_v.at[pl.ds(0, nbatch)])
        for b in range(nbatch):
            pltpu.async_copy(table_hbm.at[idx_v.at[b]],
                             rows_v.at[pl.ds(b * CHUNK, CHUNK)], sem)
        for b in range(nbatch):
            pltpu.make_async_copy(table_hbm.at[idx_v.at[b]],
                                  rows_v.at[pl.ds(b * CHUNK, CHUNK)], sem).wait()

        def row_body(g, _):
            for d in range(D // LANES):
                s = pl.ds(d * LANES, LANES)
                acc = rows_v[bag * g, s]
                for j in range(1, bag):
                    acc = acc + rows_v[bag * g + j, s]
                out_v[g, s] = acc
            return 0

        lax.fori_loop(0, CHUNK, row_body, 0, unroll=False)
        pltpu.sync_copy(out_v, out_hbm.at[pl.ds(bag0, CHUNK)])
        return 0

    lax.fori_loop(0, chunks_per_tile, chunk_body, 0, unroll=False)


def _sc_kernel(vp_bags, ep_bags, v_chunks, e_chunks):
    mesh = plsc.VectorSubcoreMesh(core_axis_name="c", subcore_axis_name="s")

    @functools.partial(
        pl.kernel,
        out_type=(
            jax.ShapeDtypeStruct((vp_bags, D), jnp.float32),
            jax.ShapeDtypeStruct((ep_bags, D), jnp.float32),
        ),
        mesh=mesh,
        scratch_types=[
            pltpu.VMEM((4, CHUNK), jnp.int32),
            pltpu.VMEM((4 * CHUNK, D), jnp.float32),
            pltpu.VMEM((CHUNK, D), jnp.float32),
            pltpu.SemaphoreType.DMA,
        ],
    )
    def k(vidx_hbm, eidx_hbm, node_hbm, edge_hbm, vout_hbm, eout_hbm,
          idx_v, rows_v, out_v, sem):
        wid = lax.axis_index("s") * NC + lax.axis_index("c")
        _embed_body(vidx_hbm, node_hbm, vout_hbm, bag=4,
                    chunks_per_tile=v_chunks, wid=wid,
                    idx_v=idx_v, rows_v=rows_v, out_v=out_v, sem=sem)
        _embed_body(eidx_hbm, edge_hbm, eout_hbm, bag=2,
                    chunks_per_tile=e_chunks, wid=wid,
                    idx_v=idx_v, rows_v=rows_v, out_v=out_v, sem=sem)

    return k


def kernel(V, E, node_table, edge_table):
    nv, bag_v = V.shape
    ne, bag_e = E.shape
    grain = NW * CHUNK
    vp = -(-nv // grain) * grain
    ep = -(-ne // grain) * grain
    v_chunks = vp // grain
    e_chunks = ep // grain

    vflat = jnp.zeros((vp * bag_v,), jnp.int32).at[: nv * bag_v].set(
        V.reshape(-1).astype(jnp.int32))
    eflat = jnp.zeros((ep * bag_e,), jnp.int32).at[: ne * bag_e].set(
        E.reshape(-1).astype(jnp.int32))
    vidx = vflat.reshape(-1, CHUNK)
    eidx = eflat.reshape(-1, CHUNK)

    vout, eout = _sc_kernel(vp, ep, v_chunks, e_chunks)(
        vidx, eidx, node_table, edge_table)
    return vout[:nv], eout[:ne]


# speedup vs baseline: 1.9941x; 1.9941x over previous
import functools

import jax
import jax.numpy as jnp
from jax import lax
from jax.experimental import pallas as pl
from jax.experimental.pallas import tpu as pltpu
from jax.experimental.pallas import tpu_sc as plsc

NC = 2
NS = 16
NW = NC * NS
LANES = 16
D = 128
CHUNK = 128


def _embed_body(idx_hbm, table_hbm, out_hbm, *, bag, chunks_per_tile, wid,
                idx_v, rows_v, out_v, sem):
    nrows = bag * CHUNK
    nbatch = nrows // CHUNK

    def chunk_body(c, _):
        bag0 = wid * (chunks_per_tile * CHUNK) + c * CHUNK
        pltpu.sync_copy(idx_hbm.at[pl.ds(bag0 * bag, nrows)],
                        idx_v.at[pl.ds(0, nrows)])
        for b in range(nbatch):
            pltpu.async_copy(table_hbm.at[idx_v.at[pl.ds(b * CHUNK, CHUNK)]],
                             rows_v.at[pl.ds(b * CHUNK, CHUNK)], sem)
        for b in range(nbatch):
            pltpu.make_async_copy(table_hbm.at[idx_v.at[pl.ds(b * CHUNK, CHUNK)]],
                                  rows_v.at[pl.ds(b * CHUNK, CHUNK)], sem).wait()

        def row_body(g, _):
            for d in range(D // LANES):
                s = pl.ds(d * LANES, LANES)
                acc = rows_v[bag * g, s]
                for j in range(1, bag):
                    acc = acc + rows_v[bag * g + j, s]
                out_v[g, s] = acc
            return 0

        lax.fori_loop(0, CHUNK, row_body, 0, unroll=False)
        pltpu.sync_copy(out_v, out_hbm.at[pl.ds(bag0, CHUNK)])
        return 0

    lax.fori_loop(0, chunks_per_tile, chunk_body, 0, unroll=False)


def _sc_kernel(vp_bags, ep_bags, v_chunks, e_chunks):
    mesh = plsc.VectorSubcoreMesh(core_axis_name="c", subcore_axis_name="s")

    @functools.partial(
        pl.kernel,
        out_type=(
            jax.ShapeDtypeStruct((vp_bags, D), jnp.float32),
            jax.ShapeDtypeStruct((ep_bags, D), jnp.float32),
        ),
        mesh=mesh,
        scratch_types=[
            pltpu.VMEM((4 * CHUNK,), jnp.int32),
            pltpu.VMEM((4 * CHUNK, D), jnp.float32),
            pltpu.VMEM((CHUNK, D), jnp.float32),
            pltpu.SemaphoreType.DMA,
        ],
    )
    def k(vidx_hbm, eidx_hbm, node_hbm, edge_hbm, vout_hbm, eout_hbm,
          idx_v, rows_v, out_v, sem):
        wid = lax.axis_index("s") * NC + lax.axis_index("c")
        _embed_body(vidx_hbm, node_hbm, vout_hbm, bag=4,
                    chunks_per_tile=v_chunks, wid=wid,
                    idx_v=idx_v, rows_v=rows_v, out_v=out_v, sem=sem)
        _embed_body(eidx_hbm, edge_hbm, eout_hbm, bag=2,
                    chunks_per_tile=e_chunks, wid=wid,
                    idx_v=idx_v, rows_v=rows_v, out_v=out_v, sem=sem)

    return k


def kernel(V, E, node_table, edge_table):
    nv, bag_v = V.shape
    ne, bag_e = E.shape
    grain = NW * CHUNK
    vp = -(-nv // grain) * grain
    ep = -(-ne // grain) * grain
    v_chunks = vp // grain
    e_chunks = ep // grain

    vflat = jnp.zeros((vp * bag_v,), jnp.int32).at[: nv * bag_v].set(
        V.reshape(-1).astype(jnp.int32))
    eflat = jnp.zeros((ep * bag_e,), jnp.int32).at[: ne * bag_e].set(
        E.reshape(-1).astype(jnp.int32))

    vout, eout = _sc_kernel(vp, ep, v_chunks, e_chunks)(
        vflat, eflat, node_table, edge_table)
    return vout[:nv], eout[:ne]


# speedup vs baseline: 2.6702x; 1.3391x over previous
import functools

import jax
import jax.numpy as jnp
from jax import lax
from jax.experimental import pallas as pl
from jax.experimental.pallas import tpu as pltpu
from jax.experimental.pallas import tpu_sc as plsc

NC = 2
NS = 16
NW = NC * NS
LANES = 16
D = 128
CHUNK = 128


def _embed_body(idx_hbm, table_hbm, out_hbm, *, bag, total, wid,
                idx_v, rows_v, out_v, sem):
    nrows = bag * CHUNK
    nbatch = nrows // CHUNK
    chunks_per_tile = -(-total // (NW * CHUNK))
    last_start = total - CHUNK

    def chunk_body(c, _):
        bag0 = jnp.minimum(wid * (chunks_per_tile * CHUNK) + c * CHUNK,
                           last_start)
        pltpu.sync_copy(idx_hbm.at[pl.ds(bag0 * bag, nrows)],
                        idx_v.at[pl.ds(0, nrows)])
        for b in range(nbatch):
            pltpu.async_copy(table_hbm.at[idx_v.at[pl.ds(b * CHUNK, CHUNK)]],
                             rows_v.at[pl.ds(b * CHUNK, CHUNK)], sem)
        for b in range(nbatch):
            pltpu.make_async_copy(table_hbm.at[idx_v.at[pl.ds(b * CHUNK, CHUNK)]],
                                  rows_v.at[pl.ds(b * CHUNK, CHUNK)], sem).wait()

        def row_body(g, _):
            for d in range(D // LANES):
                s = pl.ds(d * LANES, LANES)
                acc = rows_v[bag * g, s]
                for j in range(1, bag):
                    acc = acc + rows_v[bag * g + j, s]
                out_v[g, s] = acc
            return 0

        lax.fori_loop(0, CHUNK, row_body, 0, unroll=False)
        pltpu.sync_copy(out_v, out_hbm.at[pl.ds(bag0, CHUNK)])
        return 0

    lax.fori_loop(0, chunks_per_tile, chunk_body, 0, unroll=False)


def _sc_kernel(nv, ne):
    mesh = plsc.VectorSubcoreMesh(core_axis_name="c", subcore_axis_name="s")

    @functools.partial(
        pl.kernel,
        out_type=(
            jax.ShapeDtypeStruct((nv, D), jnp.float32),
            jax.ShapeDtypeStruct((ne, D), jnp.float32),
        ),
        mesh=mesh,
        scratch_types=[
            pltpu.VMEM((4 * CHUNK,), jnp.int32),
            pltpu.VMEM((4 * CHUNK, D), jnp.float32),
            pltpu.VMEM((CHUNK, D), jnp.float32),
            pltpu.SemaphoreType.DMA,
        ],
    )
    def k(vidx_hbm, eidx_hbm, node_hbm, edge_hbm, vout_hbm, eout_hbm,
          idx_v, rows_v, out_v, sem):
        wid = lax.axis_index("s") * NC + lax.axis_index("c")
        _embed_body(vidx_hbm, node_hbm, vout_hbm, bag=4, total=nv, wid=wid,
                    idx_v=idx_v, rows_v=rows_v, out_v=out_v, sem=sem)
        _embed_body(eidx_hbm, edge_hbm, eout_hbm, bag=2, total=ne, wid=wid,
                    idx_v=idx_v, rows_v=rows_v, out_v=out_v, sem=sem)

    return k


def kernel(V, E, node_table, edge_table):
    nv, bag_v = V.shape
    ne, bag_e = E.shape
    assert bag_v == 4 and bag_e == 2 and nv % 8 == 0 and ne % 8 == 0
    vflat = V.reshape(-1).astype(jnp.int32)
    eflat = E.reshape(-1).astype(jnp.int32)
    return _sc_kernel(nv, ne)(vflat, eflat, node_table, edge_table)


# speedup vs baseline: 4.3608x; 1.6331x over previous
import functools

import jax
import jax.numpy as jnp
from jax import lax
from jax.experimental import pallas as pl
from jax.experimental.pallas import tpu as pltpu
from jax.experimental.pallas import tpu_sc as plsc

NC = 2
NS = 16
NW = NC * NS
LANES = 16
D = 128
IB = 128
NROWS = 256


def _phase(idx_hbm, table_hbm, out_hbm, *, bag, total, wid,
           idx0, idx1, rows0, rows1, out0, out1, sg0, sg1, so0, so1):
    bpc = NROWS // bag
    nb = NROWS // IB
    n = -(-total // (NW * bpc))
    n += n & 1
    tile0 = wid * n * bpc
    last = total - bpc

    def start_of(c):
        return jnp.minimum(tile0 + c * bpc, last)

    def stage(c, idxb):
        pltpu.sync_copy(idx_hbm.at[pl.ds(start_of(c) * bag, NROWS)],
                        idxb.at[pl.ds(0, NROWS)])

    def fire(idxb, rowsb, sem):
        for b in range(nb):
            pltpu.async_copy(table_hbm.at[idxb.at[pl.ds(b * IB, IB)]],
                             rowsb.at[pl.ds(b * IB, IB)], sem)

    def drain(idxb, rowsb, sem):
        for b in range(nb):
            pltpu.make_async_copy(table_hbm.at[idxb.at[pl.ds(b * IB, IB)]],
                                  rowsb.at[pl.ds(b * IB, IB)], sem).wait()

    def compute(rowsb, outb):
        @plsc.parallel_loop(0, bpc, unroll=4)
        def _(g):
            for d in range(D // LANES):
                s = pl.ds(d * LANES, LANES)
                acc = rowsb[bag * g, s]
                for j in range(1, bag):
                    acc = acc + rowsb[bag * g + j, s]
                outb[g, s] = acc

    def put(c, outb, sem):
        pltpu.async_copy(outb.at[pl.ds(0, bpc)],
                         out_hbm.at[pl.ds(start_of(c), bpc)], sem)

    def put_wait(c, outb, sem):
        pltpu.make_async_copy(outb.at[pl.ds(0, bpc)],
                              out_hbm.at[pl.ds(start_of(c), bpc)], sem).wait()

    stage(0, idx0)
    fire(idx0, rows0, sg0)

    def outer(cc, _):
        c0 = 2 * cc
        stage(c0 + 1, idx1)
        fire(idx1, rows1, sg1)
        drain(idx0, rows0, sg0)

        @pl.when(cc > 0)
        def _():
            put_wait(c0 - 2, out0, so0)

        compute(rows0, out0)
        put(c0, out0, so0)

        @pl.when(c0 + 2 < n)
        def _():
            stage(c0 + 2, idx0)
            fire(idx0, rows0, sg0)

        drain(idx1, rows1, sg1)

        @pl.when(cc > 0)
        def _():
            put_wait(c0 - 1, out1, so1)

        compute(rows1, out1)
        put(c0 + 1, out1, so1)
        return 0

    lax.fori_loop(0, n // 2, outer, 0, unroll=False)
    put_wait(n - 2, out0, so0)
    put_wait(n - 1, out1, so1)


def _sc_kernel(nv, ne):
    mesh = plsc.VectorSubcoreMesh(core_axis_name="c", subcore_axis_name="s")

    @functools.partial(
        pl.kernel,
        out_type=(
            jax.ShapeDtypeStruct((nv, D), jnp.float32),
            jax.ShapeDtypeStruct((ne, D), jnp.float32),
        ),
        mesh=mesh,
        scratch_types=[
            pltpu.VMEM((NROWS,), jnp.int32),
            pltpu.VMEM((NROWS,), jnp.int32),
            pltpu.VMEM((NROWS, D), jnp.float32),
            pltpu.VMEM((NROWS, D), jnp.float32),
            pltpu.VMEM((IB, D), jnp.float32),
            pltpu.VMEM((IB, D), jnp.float32),
            pltpu.SemaphoreType.DMA,
            pltpu.SemaphoreType.DMA,
            pltpu.SemaphoreType.DMA,
            pltpu.SemaphoreType.DMA,
        ],
    )
    def k(vidx_hbm, eidx_hbm, node_hbm, edge_hbm, vout_hbm, eout_hbm,
          idx0, idx1, rows0, rows1, out0, out1, sg0, sg1, so0, so1):
        wid = lax.axis_index("s") * NC + lax.axis_index("c")
        bufs = dict(idx0=idx0, idx1=idx1, rows0=rows0, rows1=rows1,
                    out0=out0, out1=out1, sg0=sg0, sg1=sg1, so0=so0, so1=so1)
        _phase(vidx_hbm, node_hbm, vout_hbm, bag=4, total=nv, wid=wid, **bufs)
        _phase(eidx_hbm, edge_hbm, eout_hbm, bag=2, total=ne, wid=wid, **bufs)

    return k


def kernel(V, E, node_table, edge_table):
    nv, bag_v = V.shape
    ne, bag_e = E.shape
    assert bag_v == 4 and bag_e == 2 and nv % 8 == 0 and ne % 8 == 0
    vflat = V.reshape(-1).astype(jnp.int32)
    eflat = E.reshape(-1).astype(jnp.int32)
    return _sc_kernel(nv, ne)(vflat, eflat, node_table, edge_table)


# speedup vs baseline: 5.3457x; 1.2259x over previous
import functools

import jax
import jax.numpy as jnp
from jax import lax
from jax.experimental import pallas as pl
from jax.experimental.pallas import tpu as pltpu
from jax.experimental.pallas import tpu_sc as plsc

NC = 2
NS = 16
NW = NC * NS
LANES = 16
D = 128
IB = 128
NROWS = 256


def _phase(idx_hbm, table_hbm, out_hbm, *, bag, total, wid,
           idx0, idx1, rows0, rows1, out0, out1, sg0, sg1, so0, so1):
    bpc = NROWS // bag
    nb = NROWS // IB
    n = -(-total // (NW * bpc))
    n += n & 1
    tile0 = wid * n * bpc
    last = total - bpc

    def start_of(c):
        return jnp.minimum(tile0 + c * bpc, last)

    def stage(c, idxb):
        pltpu.sync_copy(idx_hbm.at[pl.ds(start_of(c) * bag, NROWS)],
                        idxb.at[pl.ds(0, NROWS)])

    def fire(idxb, rowsb, sem):
        for b in range(nb):
            pltpu.async_copy(table_hbm.at[idxb.at[pl.ds(b * IB, IB)]],
                             rowsb.at[pl.ds(b * IB, IB)], sem)

    def drain(idxb, rowsb, sem):
        for b in range(nb):
            pltpu.make_async_copy(table_hbm.at[idxb.at[pl.ds(b * IB, IB)]],
                                  rowsb.at[pl.ds(b * IB, IB)], sem).wait()

    def compute(rowsb, outb):
        @plsc.parallel_loop(0, bpc, unroll=4)
        def _(g):
            for d in range(D // LANES):
                s = pl.ds(d * LANES, LANES)
                acc = rowsb[bag * g, s]
                for j in range(1, bag):
                    acc = acc + rowsb[bag * g + j, s]
                outb[g, s] = acc

    def put(c, outb, sem):
        pltpu.async_copy(outb.at[pl.ds(0, bpc)],
                         out_hbm.at[pl.ds(start_of(c), bpc)], sem)

    def put_wait(c, outb, sem):
        pltpu.make_async_copy(outb.at[pl.ds(0, bpc)],
                              out_hbm.at[pl.ds(start_of(c), bpc)], sem).wait()

    stage(0, idx0)
    fire(idx0, rows0, sg0)

    def outer(cc, _):
        c0 = 2 * cc
        stage(c0 + 1, idx1)
        fire(idx1, rows1, sg1)
        drain(idx0, rows0, sg0)

        @pl.when(cc > 0)
        def _():
            put_wait(c0 - 2, out0, so0)

        compute(rows0, out0)
        put(c0, out0, so0)

        @pl.when(c0 + 2 < n)
        def _():
            stage(c0 + 2, idx0)
            fire(idx0, rows0, sg0)

        drain(idx1, rows1, sg1)

        @pl.when(cc > 0)
        def _():
            put_wait(c0 - 1, out1, so1)

        compute(rows1, out1)
        put(c0 + 1, out1, so1)
        return 0

    lax.fori_loop(0, n // 2, outer, 0, unroll=False)
    put_wait(n - 2, out0, so0)
    put_wait(n - 1, out1, so1)


def _sc_kernel(nv, ne):
    mesh = plsc.VectorSubcoreMesh(core_axis_name="c", subcore_axis_name="s")

    @functools.partial(
        pl.kernel,
        out_type=(
            jax.ShapeDtypeStruct((nv, D), jnp.float32),
            jax.ShapeDtypeStruct((ne, D), jnp.float32),
        ),
        mesh=mesh,
        scratch_types=[
            pltpu.VMEM((NROWS,), jnp.int32),
            pltpu.VMEM((NROWS,), jnp.int32),
            pltpu.VMEM((NROWS, D), jnp.float32),
            pltpu.VMEM((NROWS, D), jnp.float32),
            pltpu.VMEM((IB, D), jnp.float32),
            pltpu.VMEM((IB, D), jnp.float32),
            pltpu.SemaphoreType.DMA,
            pltpu.SemaphoreType.DMA,
            pltpu.SemaphoreType.DMA,
            pltpu.SemaphoreType.DMA,
            pltpu.VMEM_SHARED((1000, D), jnp.float32),
        ],
    )
    def k(vidx_hbm, eidx_hbm, node_hbm, edge_hbm, vout_hbm, eout_hbm,
          idx0, idx1, rows0, rows1, out0, out1, sg0, sg1, so0, so1, etab_sp):
        wid = lax.axis_index("s") * NC + lax.axis_index("c")
        sid = lax.axis_index("s")
        bufs = dict(idx0=idx0, idx1=idx1, rows0=rows0, rows1=rows1,
                    out0=out0, out1=out1, sg0=sg0, sg1=sg1, so0=so0, so1=so1)

        @pl.when(sid < 7)
        def _():
            pltpu.sync_copy(edge_hbm.at[pl.ds(sid * 128, 128)],
                            etab_sp.at[pl.ds(sid * 128, 128)])

        @pl.when(sid == 7)
        def _():
            pltpu.sync_copy(edge_hbm.at[pl.ds(896, 104)],
                            etab_sp.at[pl.ds(896, 104)])

        _phase(vidx_hbm, node_hbm, vout_hbm, bag=4, total=nv, wid=wid, **bufs)
        plsc.subcore_barrier()
        _phase(eidx_hbm, etab_sp, eout_hbm, bag=2, total=ne, wid=wid, **bufs)

    return k


def kernel(V, E, node_table, edge_table):
    nv, bag_v = V.shape
    ne, bag_e = E.shape
    assert bag_v == 4 and bag_e == 2 and nv % 8 == 0 and ne % 8 == 0
    vflat = V.reshape(-1).astype(jnp.int32)
    eflat = E.reshape(-1).astype(jnp.int32)
    return _sc_kernel(nv, ne)(vflat, eflat, node_table, edge_table)


# speedup vs baseline: 16.1410x; 3.0195x over previous
import functools

import jax
import jax.numpy as jnp
from jax import lax
from jax.experimental import pallas as pl
from jax.experimental.pallas import tpu as pltpu
from jax.experimental.pallas import tpu_sc as plsc

NC = 2
NS = 16
NW = NC * NS
LANES = 16
D = 128
NROWS = 256


def _phase(idx_hbm, table_hbm, out_hbm, *, bag, total, wid, slots):
    bpc = NROWS // bag
    n = -(-total // (NW * bpc))
    n = -(-n // 3) * 3
    tile0 = wid * n * bpc
    last = total - bpc

    def start_of(c):
        return jnp.minimum(tile0 + c * bpc, last)

    def stage(c, idxb, sem):
        s0 = start_of(c)
        for j in range(bag):
            pltpu.async_copy(idx_hbm.at[pl.ds(j * total + s0, bpc)],
                             idxb.at[pl.ds(j * bpc, bpc)], sem)

    def stage_wait(c, idxb, sem):
        s0 = start_of(c)
        for j in range(bag):
            pltpu.make_async_copy(idx_hbm.at[pl.ds(j * total + s0, bpc)],
                                  idxb.at[pl.ds(j * bpc, bpc)], sem).wait()

    def fire(idxb, rowsb, sem):
        for j in range(bag):
            pltpu.async_copy(table_hbm.at[idxb.at[pl.ds(j * bpc, bpc)]],
                             rowsb.at[pl.ds(j * bpc, bpc)], sem)

    def drain(idxb, rowsb, sem):
        for j in range(bag):
            pltpu.make_async_copy(table_hbm.at[idxb.at[pl.ds(j * bpc, bpc)]],
                                  rowsb.at[pl.ds(j * bpc, bpc)], sem).wait()

    def compute(rowsb):
        @plsc.parallel_loop(0, bpc, unroll=8)
        def _(g):
            for d in range(D // LANES):
                s = pl.ds(d * LANES, LANES)
                acc = rowsb[bpc + g, s]
                for j in range(2, bag):
                    acc = acc + rowsb[j * bpc + g, s]
                plsc.addupdate(rowsb.at[g, s], acc)

    def put(c, rowsb, sem):
        pltpu.async_copy(rowsb.at[pl.ds(0, bpc)],
                         out_hbm.at[pl.ds(start_of(c), bpc)], sem)

    def put_wait(c, rowsb, sem):
        pltpu.make_async_copy(rowsb.at[pl.ds(0, bpc)],
                              out_hbm.at[pl.ds(start_of(c), bpc)], sem).wait()

    for k in range(3):
        stage(k, slots[k][0], slots[k][3])
    stage_wait(0, slots[0][0], slots[0][3])
    fire(slots[0][0], slots[0][1], slots[0][2])

    def step(c, X, Y):
        idxX, rowsX, sgX, siX, soX = X
        idxY, rowsY, sgY, siY, soY = Y
        drain(idxX, rowsX, sgX)

        @pl.when(c + 3 < n)
        def _():
            stage(c + 3, idxX, siX)

        @pl.when(c + 1 < n)
        def _():
            stage_wait(c + 1, idxY, siY)

            @pl.when(c >= 2)
            def _():
                put_wait(c - 2, rowsY, soY)

            fire(idxY, rowsY, sgY)

        compute(rowsX)
        put(c, rowsX, soX)

    def outer(cc, _):
        c0 = 3 * cc
        for k in range(3):
            step(c0 + k, slots[k], slots[(k + 1) % 3])
        return 0

    lax.fori_loop(0, n // 3, outer, 0, unroll=False)
    for c in (n - 3, n - 2, n - 1):
        put_wait(c, slots[c % 3][1], slots[c % 3][4])


def _sc_kernel(nv, ne):
    mesh = plsc.VectorSubcoreMesh(core_axis_name="c", subcore_axis_name="s")

    @functools.partial(
        pl.kernel,
        out_type=(
            jax.ShapeDtypeStruct((nv, D), jnp.float32),
            jax.ShapeDtypeStruct((ne, D), jnp.float32),
        ),
        mesh=mesh,
        compiler_params=pltpu.CompilerParams(use_tc_tiling_on_sc=True),
        scratch_types=[
            pltpu.VMEM((NROWS,), jnp.int32),
            pltpu.VMEM((NROWS,), jnp.int32),
            pltpu.VMEM((NROWS,), jnp.int32),
            pltpu.VMEM((NROWS, D), jnp.float32),
            pltpu.VMEM((NROWS, D), jnp.float32),
            pltpu.VMEM((NROWS, D), jnp.float32),
            pltpu.SemaphoreType.DMA,
            pltpu.SemaphoreType.DMA,
            pltpu.SemaphoreType.DMA,
            pltpu.SemaphoreType.DMA,
            pltpu.SemaphoreType.DMA,
            pltpu.SemaphoreType.DMA,
            pltpu.SemaphoreType.DMA,
            pltpu.SemaphoreType.DMA,
            pltpu.SemaphoreType.DMA,
            pltpu.VMEM_SHARED((1000, D), jnp.float32),
        ],
    )
    def k(vidx_hbm, eidx_hbm, node_hbm, edge_hbm, vout_hbm, eout_hbm,
          idx0, idx1, idx2, rows0, rows1, rows2,
          sg0, sg1, sg2, si0, si1, si2, so0, so1, so2, etab_sp):
        wid = lax.axis_index("s") * NC + lax.axis_index("c")
        sid = lax.axis_index("s")
        slots = ((idx0, rows0, sg0, si0, so0),
                 (idx1, rows1, sg1, si1, so1),
                 (idx2, rows2, sg2, si2, so2))

        @pl.when(sid < 7)
        def _():
            pltpu.sync_copy(edge_hbm.at[pl.ds(sid * 128, 128)],
                            etab_sp.at[pl.ds(sid * 128, 128)])

        @pl.when(sid == 7)
        def _():
            pltpu.sync_copy(edge_hbm.at[pl.ds(896, 104)],
                            etab_sp.at[pl.ds(896, 104)])

        _phase(vidx_hbm, node_hbm, vout_hbm, bag=4, total=nv, wid=wid,
               slots=slots)
        plsc.subcore_barrier()
        _phase(eidx_hbm, etab_sp, eout_hbm, bag=2, total=ne, wid=wid,
               slots=slots)

    return k


def kernel(V, E, node_table, edge_table):
    nv, bag_v = V.shape
    ne, bag_e = E.shape
    assert bag_v == 4 and bag_e == 2 and nv % 8 == 0 and ne % 8 == 0
    vflat = jnp.concatenate([V[:, j].astype(jnp.int32) for j in range(bag_v)])
    eflat = jnp.concatenate([E[:, j].astype(jnp.int32) for j in range(bag_e)])
    return _sc_kernel(nv, ne)(vflat, eflat, node_table, edge_table)
